# baseline (speedup 1.0000x reference)
import functools

import jax
import jax.numpy as jnp
from jax import lax
from jax.experimental import pallas as pl
from jax.experimental.pallas import tpu as pltpu
from jax.experimental.pallas import tpu_sc as plsc

LATENT_DIM = 128
BATCH = 16384
NC = 2
NS = 16
NW = NC * NS
B_PER_W = BATCH // NW
CHUNK = 128
NCHUNK = B_PER_W // CHUNK
NTASK = 2 * NCHUNK
NB = 6


def _sc_gather_body(sidx_hbm, eidx_hbm, s_emb, d_emb, disc_flat,
                    out_s, out_d, out_disc, *scratch):
    sidx_v, eidx_v = scratch[0], scratch[1]
    bufs = list(scratch[2:2 + NB])
    disc_v = scratch[2 + NB]
    gs = list(scratch[3 + NB:3 + 2 * NB])
    ws = list(scratch[3 + 2 * NB:3 + 3 * NB])
    dsem = scratch[3 + 3 * NB]

    wid = lax.axis_index("s") * NC + lax.axis_index("c")
    ibase = wid * NCHUNK
    obase = wid * B_PER_W
    for j in range(NCHUNK):
        pltpu.sync_copy(sidx_hbm.at[pl.ds(obase + j * CHUNK, CHUNK)],
                        sidx_v.at[j])
        pltpu.sync_copy(eidx_hbm.at[pl.ds(obase + j * CHUNK, CHUNK)],
                        eidx_v.at[j])

    def task(i):
        if i < NCHUNK:
            return s_emb, sidx_v.at[i], out_s, obase + i * CHUNK
        j = i - NCHUNK
        return d_emb, eidx_v.at[j], out_d, obase + j * CHUNK

    dcops = [pltpu.async_copy(disc_flat.at[eidx_v.at[j]], disc_v.at[j], dsem)
             for j in range(NCHUNK)]

    gops = [None] * NTASK
    wops = [None] * NTASK
    for i in range(NB):
        tbl, idx, _, _ = task(i)
        gops[i] = pltpu.async_copy(tbl.at[idx], bufs[i], gs[i])
    for i in range(NTASK):
        b = i % NB
        gops[i].wait()
        _, _, out, row = task(i)
        wops[i] = pltpu.async_copy(bufs[b], out.at[pl.ds(row, CHUNK)], ws[b])
        nxt = i + NB
        if nxt < NTASK:
            wops[i].wait()
            tbl, idx, _, _ = task(nxt)
            gops[nxt] = pltpu.async_copy(tbl.at[idx], bufs[b], gs[b])
    for j in range(NCHUNK):
        dcops[j].wait()
    pltpu.sync_copy(disc_v, out_disc.at[pl.ds(ibase, NCHUNK)])
    for i in range(NTASK):
        if i + NB >= NTASK:
            wops[i].wait()


@jax.jit
def _sc_gather(sidx, eidx, student_emb, diff_emb, disc_flat):
    mesh = plsc.VectorSubcoreMesh(core_axis_name="c", subcore_axis_name="s")
    scratch = (
        [pltpu.VMEM((NCHUNK, CHUNK), jnp.int32),
         pltpu.VMEM((NCHUNK, CHUNK), jnp.int32)]
        + [pltpu.VMEM((CHUNK, LATENT_DIM), jnp.float32) for _ in range(NB)]
        + [pltpu.VMEM((NCHUNK, CHUNK), jnp.float32)]
        + [pltpu.SemaphoreType.DMA for _ in range(2 * NB + 1)]
    )
    f = functools.partial(
        pl.kernel,
        mesh=mesh,
        out_type=[
            jax.ShapeDtypeStruct((BATCH, LATENT_DIM), jnp.float32),
            jax.ShapeDtypeStruct((BATCH, LATENT_DIM), jnp.float32),
            jax.ShapeDtypeStruct((BATCH // CHUNK, CHUNK), jnp.float32),
        ],
        scratch_types=scratch,
    )(_sc_gather_body)
    return f(sidx, eidx, student_emb, diff_emb, disc_flat)


def kernel(student_id, exercise_id, q_mask, student_emb, knowledge_emb,
           diff_emb, disc_emb):
    student_ts, diff_ts, disc2d = _sc_gather(
        student_id.astype(jnp.int32), exercise_id.astype(jnp.int32),
        student_emb, diff_emb, disc_emb.reshape(-1))
    return (student_ts, diff_ts, disc2d.reshape(BATCH, 1), knowledge_emb)


# speedup vs baseline: 1.1009x; 1.1009x over previous
import functools

import jax
import jax.numpy as jnp
from jax import lax
from jax.experimental import pallas as pl
from jax.experimental.pallas import tpu as pltpu
from jax.experimental.pallas import tpu_sc as plsc

LATENT_DIM = 128
BATCH = 16384
NC = 2
NS = 16
NW = NC * NS
B_PER_W = BATCH // NW
CHUNK = 128
NCHUNK = B_PER_W // CHUNK
NTASK = 2 * NCHUNK
NB = 6


def _sc_gather_body(sidx_hbm, eidx_hbm, s_emb, d_emb, disc_flat,
                    out_s, out_d, out_disc, *scratch):
    sidx_v, eidx_v = scratch[0], scratch[1]
    bufs = list(scratch[2:2 + NB])
    disc_v = scratch[2 + NB]
    gs = list(scratch[3 + NB:3 + 2 * NB])
    ws = list(scratch[3 + 2 * NB:3 + 3 * NB])
    dsem = scratch[3 + 3 * NB]

    wid = lax.axis_index("s") * NC + lax.axis_index("c")
    ibase = wid * NCHUNK
    obase = wid * B_PER_W
    iops = []
    for j in range(NCHUNK):
        iops.append(pltpu.async_copy(
            sidx_hbm.at[pl.ds(obase + j * CHUNK, CHUNK)], sidx_v.at[j], dsem))
        iops.append(pltpu.async_copy(
            eidx_hbm.at[pl.ds(obase + j * CHUNK, CHUNK)], eidx_v.at[j], dsem))
    for op in iops:
        op.wait()

    def task(i):
        if i < NCHUNK:
            return s_emb, sidx_v.at[i], out_s, obase + i * CHUNK
        j = i - NCHUNK
        return d_emb, eidx_v.at[j], out_d, obase + j * CHUNK

    dcops = [pltpu.async_copy(disc_flat.at[eidx_v.at[j]], disc_v.at[j], dsem)
             for j in range(NCHUNK)]

    gops = [None] * NTASK
    wops = [None] * NTASK
    for i in range(NB):
        tbl, idx, _, _ = task(i)
        gops[i] = pltpu.async_copy(tbl.at[idx], bufs[i], gs[i])
    for i in range(NTASK):
        b = i % NB
        gops[i].wait()
        _, _, out, row = task(i)
        wops[i] = pltpu.async_copy(bufs[b], out.at[pl.ds(row, CHUNK)], ws[b])
        nxt = i + NB
        if nxt < NTASK:
            wops[i].wait()
            tbl, idx, _, _ = task(nxt)
            gops[nxt] = pltpu.async_copy(tbl.at[idx], bufs[b], gs[b])
    for j in range(NCHUNK):
        dcops[j].wait()
    pltpu.sync_copy(disc_v, out_disc.at[pl.ds(ibase, NCHUNK)])
    for i in range(NTASK):
        if i + NB >= NTASK:
            wops[i].wait()


@jax.jit
def _sc_gather(sidx, eidx, student_emb, diff_emb, disc_flat):
    mesh = plsc.VectorSubcoreMesh(core_axis_name="c", subcore_axis_name="s")
    scratch = (
        [pltpu.VMEM((NCHUNK, CHUNK), jnp.int32),
         pltpu.VMEM((NCHUNK, CHUNK), jnp.int32)]
        + [pltpu.VMEM((CHUNK, LATENT_DIM), jnp.float32) for _ in range(NB)]
        + [pltpu.VMEM((NCHUNK, CHUNK), jnp.float32)]
        + [pltpu.SemaphoreType.DMA for _ in range(2 * NB + 1)]
    )
    f = functools.partial(
        pl.kernel,
        mesh=mesh,
        out_type=[
            jax.ShapeDtypeStruct((BATCH, LATENT_DIM), jnp.float32),
            jax.ShapeDtypeStruct((BATCH, LATENT_DIM), jnp.float32),
            jax.ShapeDtypeStruct((BATCH // CHUNK, CHUNK), jnp.float32),
        ],
        scratch_types=scratch,
    )(_sc_gather_body)
    return f(sidx, eidx, student_emb, diff_emb, disc_flat)


def kernel(student_id, exercise_id, q_mask, student_emb, knowledge_emb,
           diff_emb, disc_emb):
    student_ts, diff_ts, disc2d = _sc_gather(
        student_id.astype(jnp.int32), exercise_id.astype(jnp.int32),
        student_emb, diff_emb, disc_emb.reshape(-1))
    return (student_ts, diff_ts, disc2d.reshape(BATCH, 1), knowledge_emb)


# speedup vs baseline: 1.1130x; 1.0110x over previous
import functools

import jax
import jax.numpy as jnp
from jax import lax
from jax.experimental import pallas as pl
from jax.experimental.pallas import tpu as pltpu
from jax.experimental.pallas import tpu_sc as plsc

LATENT_DIM = 128
BATCH = 16384
NC = 2
NS = 16
NW = NC * NS
B_PER_W = BATCH // NW
CHUNK = 128
NCHUNK = B_PER_W // CHUNK
NTASK = 2 * NCHUNK
NB = 7


def _sc_gather_body(sidx_hbm, eidx_hbm, s_emb, d_emb, disc_flat,
                    out_s, out_d, out_disc, *scratch):
    sidx_v, eidx_v = scratch[0], scratch[1]
    bufs = list(scratch[2:2 + NB])
    disc_v = scratch[2 + NB]
    gs = list(scratch[3 + NB:3 + 2 * NB])
    ws = list(scratch[3 + 2 * NB:3 + 3 * NB])
    dsem = scratch[3 + 3 * NB]

    wid = lax.axis_index("s") * NC + lax.axis_index("c")
    ibase = wid * NCHUNK
    obase = wid * B_PER_W
    iops = []
    for j in range(NCHUNK):
        iops.append(pltpu.async_copy(
            sidx_hbm.at[pl.ds(obase + j * CHUNK, CHUNK)], sidx_v.at[j], dsem))
        iops.append(pltpu.async_copy(
            eidx_hbm.at[pl.ds(obase + j * CHUNK, CHUNK)], eidx_v.at[j], dsem))
    for op in iops:
        op.wait()

    def task(i):
        if i < NCHUNK:
            return s_emb, sidx_v.at[i], out_s, obase + i * CHUNK
        j = i - NCHUNK
        return d_emb, eidx_v.at[j], out_d, obase + j * CHUNK

    gops = [None] * NTASK
    wops = [None] * NTASK
    for i in range(NB):
        tbl, idx, _, _ = task(i)
        gops[i] = pltpu.async_copy(tbl.at[idx], bufs[i], gs[i])

    dcops = [pltpu.async_copy(disc_flat.at[eidx_v.at[j]], disc_v.at[j], dsem)
             for j in range(NCHUNK)]
    for i in range(NTASK):
        b = i % NB
        gops[i].wait()
        _, _, out, row = task(i)
        wops[i] = pltpu.async_copy(bufs[b], out.at[pl.ds(row, CHUNK)], ws[b])
        nxt = i + NB
        if nxt < NTASK:
            wops[i].wait()
            tbl, idx, _, _ = task(nxt)
            gops[nxt] = pltpu.async_copy(tbl.at[idx], bufs[b], gs[b])
    for j in range(NCHUNK):
        dcops[j].wait()
    pltpu.sync_copy(disc_v, out_disc.at[pl.ds(ibase, NCHUNK)])
    for i in range(NTASK):
        if i + NB >= NTASK:
            wops[i].wait()


@jax.jit
def _sc_gather(sidx, eidx, student_emb, diff_emb, disc_flat):
    mesh = plsc.VectorSubcoreMesh(core_axis_name="c", subcore_axis_name="s")
    scratch = (
        [pltpu.VMEM((NCHUNK, CHUNK), jnp.int32),
         pltpu.VMEM((NCHUNK, CHUNK), jnp.int32)]
        + [pltpu.VMEM((CHUNK, LATENT_DIM), jnp.float32) for _ in range(NB)]
        + [pltpu.VMEM((NCHUNK, CHUNK), jnp.float32)]
        + [pltpu.SemaphoreType.DMA for _ in range(2 * NB + 1)]
    )
    f = functools.partial(
        pl.kernel,
        mesh=mesh,
        out_type=[
            jax.ShapeDtypeStruct((BATCH, LATENT_DIM), jnp.float32),
            jax.ShapeDtypeStruct((BATCH, LATENT_DIM), jnp.float32),
            jax.ShapeDtypeStruct((BATCH // CHUNK, CHUNK), jnp.float32),
        ],
        scratch_types=scratch,
    )(_sc_gather_body)
    return f(sidx, eidx, student_emb, diff_emb, disc_flat)


def kernel(student_id, exercise_id, q_mask, student_emb, knowledge_emb,
           diff_emb, disc_emb):
    student_ts, diff_ts, disc2d = _sc_gather(
        student_id.astype(jnp.int32), exercise_id.astype(jnp.int32),
        student_emb, diff_emb, disc_emb.reshape(-1))
    return (student_ts, diff_ts, disc2d.reshape(BATCH, 1), knowledge_emb)
